# baseline (speedup 1.0000x reference)
import functools

import jax
import jax.numpy as jnp
from jax import lax
from jax.experimental import pallas as pl
from jax.experimental.pallas import tpu as pltpu
from jax.experimental.pallas import tpu_sc as plsc

F = 26
V = 100000
D = 32
B = 16384
ND = 13

NC = 2
NS = 16
NW = NC * NS
CHUNK = 1024
NCH = B // CHUNK
NSC_F = 13

BT = 1024


IDX_STAGE = F * B // NS


def _sc_gather(idx, deep_t, wide_t):
  mesh = plsc.VectorSubcoreMesh(core_axis_name="c", subcore_axis_name="s")

  @functools.partial(
      pl.kernel,
      out_type=(
          jax.ShapeDtypeStruct((F * D, B), jnp.float32),
          jax.ShapeDtypeStruct((F, B), jnp.float32),
      ),
      mesh=mesh,
      scratch_types=[
          pltpu.VMEM((1, V), jnp.float32),
          pltpu.VMEM((CHUNK,), jnp.int32),
          pltpu.VMEM((CHUNK,), jnp.int32),
          pltpu.VMEM((1, CHUNK), jnp.float32),
          pltpu.VMEM((1, CHUNK), jnp.float32),
          pltpu.SemaphoreType.DMA,
          pltpu.SemaphoreType.DMA,
          pltpu.SemaphoreType.DMA,
          pltpu.SemaphoreType.DMA,
          pltpu.SemaphoreType.DMA,
          pltpu.VMEM_SHARED((F * B,), jnp.int32),
      ],
      compiler_params=pltpu.CompilerParams(use_tc_tiling_on_sc=True,
                                           needs_layout_passes=False),
  )
  def k(idx_hbm, deep_hbm, wide_hbm, emb_out, wide_out,
        row_v, idx0_v, idx1_v, g0_v, g1_v,
        rsem, isem0, isem1, osem0, osem1, idx_sh):
    c = lax.axis_index("c")
    s = lax.axis_index("s")
    w = s * NC + c

    pltpu.sync_copy(idx_hbm.at[pl.ds(s * IDX_STAGE, IDX_STAGE)],
                    idx_sh.at[pl.ds(s * IDX_STAGE, IDX_STAGE)])
    plsc.subcore_barrier()

    zero16 = jnp.zeros((16,), jnp.int32)
    idxb = (idx0_v, idx1_v)
    goutb = (g0_v, g1_v)
    isems = (isem0, isem1)
    osems = (osem0, osem1)

    def gather_chunk(idxc_v, gout_v):
      def g(i, carry):
        for u in range(8):
          sl = pl.ds((i * 8 + u) * 16, 16)
          gout_v[0, sl] = plsc.load_gather(row_v, [zero16, idxc_v[sl]])
        return carry
      lax.fori_loop(0, CHUNK // (16 * 8), g, 0)

    def row_pipeline(src_hbm, src_row, idx_row, out_hbm, out_row):
      def idx_slice(h):
        return idx_sh.at[pl.ds(idx_row * B + h * CHUNK, CHUNK)]

      def out_slice(h):
        return out_hbm.at[pl.ds(out_row, 1), pl.ds(h * CHUNK, CHUNK)]

      pltpu.async_copy(src_hbm.at[pl.ds(src_row, 1)], row_v, rsem)
      pltpu.async_copy(idx_slice(0), idxb[0], isems[0])
      pltpu.async_copy(idx_slice(1), idxb[1], isems[1])
      pltpu.make_async_copy(src_hbm.at[pl.ds(src_row, 1)], row_v,
                            rsem).wait()
      for h in range(NCH):
        b = h % 2
        pltpu.make_async_copy(idx_slice(h), idxb[b], isems[b]).wait()
        if h >= 2:
          pltpu.make_async_copy(goutb[b], out_slice(h - 2), osems[b]).wait()
        gather_chunk(idxb[b], goutb[b])
        if h + 2 < NCH:
          pltpu.async_copy(idx_slice(h + 2), idxb[b], isems[b])
        pltpu.async_copy(goutb[b], out_slice(h), osems[b])
      for h in (NCH - 2, NCH - 1):
        b = h % 2
        pltpu.make_async_copy(goutb[b], out_slice(h), osems[b]).wait()

    def deep_body(f, carry):
      row_pipeline(deep_hbm, f * D + w, f, emb_out, f * D + w)
      return carry

    lax.fori_loop(0, F, deep_body, 0)

    @pl.when(w < F)
    def _wide():
      row_pipeline(wide_hbm, w, w, wide_out, w)

  return k(idx, deep_t, wide_t)


def _tc_mlp(emb_t, dense_t, wide2, dwt, db, w1e, w1d, b1, w2, b2, w3, b3,
            wwt, wb, bias):
  c0 = (((0,), (0,)), ((), ()))

  def body(emb_ref, dense_ref, ws_ref, dwt_ref, db_ref, w1e_ref, w1d_ref,
           b1_ref, w2_ref, b2_ref, w3_ref, b3_ref, wwt_ref, wb_ref,
           bias_ref, out_ref):
    dense_blk = dense_ref[...]
    dd = lax.dot_general(dense_blk, dwt_ref[...], c0,
                         preferred_element_type=jnp.float32) + db_ref[...]
    h1 = lax.dot_general(emb_ref[...], w1e_ref[...], c0,
                         preferred_element_type=jnp.float32)
    h1 = h1 + jnp.dot(dd, w1d_ref[...],
                      preferred_element_type=jnp.float32) + b1_ref[...]
    h1 = jnp.maximum(h1, 0.0)
    h2 = jnp.maximum(
        jnp.dot(h1, w2_ref[...], preferred_element_type=jnp.float32)
        + b2_ref[...], 0.0)
    h3 = jnp.maximum(
        jnp.dot(h2, w3_ref[...], preferred_element_type=jnp.float32)
        + b3_ref[...], 0.0)
    wd = lax.dot_general(dense_blk, wwt_ref[...], c0,
                         preferred_element_type=jnp.float32) + wb_ref[...]
    ws = lax.dot_general(ws_ref[...], jnp.ones((F, 1), jnp.float32), c0,
                         preferred_element_type=jnp.float32)
    out_ref[...] = bias_ref[...] + ws + wd + h3

  full = lambda a: pl.BlockSpec(a.shape, lambda i: (0,) * a.ndim)
  col_spec = lambda rows: pl.BlockSpec((rows, BT), lambda i: (0, i))
  return pl.pallas_call(
      body,
      grid=(B // BT,),
      in_specs=[
          col_spec(F * D),
          col_spec(ND),
          col_spec(F),
          full(dwt), full(db), full(w1e), full(w1d), full(b1),
          full(w2), full(b2), full(w3), full(b3),
          full(wwt), full(wb), full(bias),
      ],
      out_specs=pl.BlockSpec((BT, 1), lambda i: (i, 0)),
      out_shape=jax.ShapeDtypeStruct((B, 1), jnp.float32),
  )(emb_t, dense_t, wide2, dwt, db, w1e, w1d, b1, w2, b2, w3, b3,
    wwt, wb, bias)


def kernel(sparse_features, dense_features, wide_emb, wide_w, wide_b,
           deep_emb, deep_w, deep_b, W1, b1, W2, b2, W3, b3, bias):
  deep_t = deep_emb.transpose(0, 2, 1).reshape(F * D, V)
  wide_t = wide_emb.reshape(F, V)

  emb_t, wide2 = _sc_gather(sparse_features.reshape(F * B), deep_t, wide_t)

  return _tc_mlp(
      emb_t,
      dense_features.T,
      wide2,
      deep_w.T,
      deep_b.reshape(1, D),
      W1[:, D:].T,
      W1[:, :D].T,
      b1.reshape(1, 512),
      W2.T,
      b2.reshape(1, 256),
      W3.T,
      b3.reshape(1, 1),
      wide_w.T,
      wide_b.reshape(1, 1),
      bias,
  )


# speedup vs baseline: 1.0654x; 1.0654x over previous
import functools

import jax
import jax.numpy as jnp
from jax import lax
from jax.experimental import pallas as pl
from jax.experimental.pallas import tpu as pltpu
from jax.experimental.pallas import tpu_sc as plsc

F = 26
V = 100000
D = 32
B = 16384
ND = 13

NC = 2
NS = 16
NW = NC * NS
CHUNK = 4096
NCH = B // CHUNK
NSC_F = 13

BT = 1024


def _sc_gather(idx, deep_t, wide_t):
  mesh = plsc.VectorSubcoreMesh(core_axis_name="c", subcore_axis_name="s")

  @functools.partial(
      pl.kernel,
      out_type=(
          jax.ShapeDtypeStruct((F * D, B), jnp.float32),
          jax.ShapeDtypeStruct((F, B), jnp.float32),
      ),
      mesh=mesh,
      scratch_types=[
          pltpu.VMEM((1, V), jnp.float32),
          pltpu.VMEM((1, CHUNK), jnp.int32),
          pltpu.VMEM((1, CHUNK), jnp.int32),
          pltpu.VMEM((1, CHUNK), jnp.float32),
          pltpu.VMEM((1, CHUNK), jnp.float32),
          pltpu.SemaphoreType.DMA,
          pltpu.SemaphoreType.DMA,
          pltpu.SemaphoreType.DMA,
          pltpu.SemaphoreType.DMA,
          pltpu.SemaphoreType.DMA,
      ],
      compiler_params=pltpu.CompilerParams(use_tc_tiling_on_sc=True,
                                           needs_layout_passes=False),
  )
  def k(idx_hbm, deep_hbm, wide_hbm, emb_out, wide_out,
        row_v, idx0_v, idx1_v, g0_v, g1_v,
        rsem, isem0, isem1, osem0, osem1):
    c = lax.axis_index("c")
    s = lax.axis_index("s")
    w = s * NC + c

    zero16 = jnp.zeros((16,), jnp.int32)
    idxb = (idx0_v, idx1_v)
    goutb = (g0_v, g1_v)
    isems = (isem0, isem1)
    osems = (osem0, osem1)

    def gather_chunk(idxc_v, gout_v):
      def g(i, carry):
        for u in range(8):
          sl = pl.ds((i * 8 + u) * 16, 16)
          gout_v[0, sl] = plsc.load_gather(row_v, [zero16, idxc_v[0, sl]])
        return carry
      lax.fori_loop(0, CHUNK // (16 * 8), g, 0)

    def row_pipeline(src_hbm, src_row, idx_row, out_hbm, out_row):
      def idx_slice(h):
        return idx_hbm.at[pl.ds(idx_row, 1), pl.ds(h * CHUNK, CHUNK)]

      def out_slice(h):
        return out_hbm.at[pl.ds(out_row, 1), pl.ds(h * CHUNK, CHUNK)]

      pltpu.async_copy(src_hbm.at[pl.ds(src_row, 1)], row_v, rsem)
      pltpu.async_copy(idx_slice(0), idxb[0], isems[0])
      pltpu.async_copy(idx_slice(1), idxb[1], isems[1])
      pltpu.make_async_copy(src_hbm.at[pl.ds(src_row, 1)], row_v,
                            rsem).wait()
      for h in range(NCH):
        b = h % 2
        pltpu.make_async_copy(idx_slice(h), idxb[b], isems[b]).wait()
        if h >= 2:
          pltpu.make_async_copy(goutb[b], out_slice(h - 2), osems[b]).wait()
        gather_chunk(idxb[b], goutb[b])
        if h + 2 < NCH:
          pltpu.async_copy(idx_slice(h + 2), idxb[b], isems[b])
        pltpu.async_copy(goutb[b], out_slice(h), osems[b])
      for h in (NCH - 2, NCH - 1):
        b = h % 2
        pltpu.make_async_copy(goutb[b], out_slice(h), osems[b]).wait()

    def deep_body(f, carry):
      row_pipeline(deep_hbm, f * D + w, f, emb_out, f * D + w)
      return carry

    lax.fori_loop(0, F, deep_body, 0)

    @pl.when(w < F)
    def _wide():
      row_pipeline(wide_hbm, w, w, wide_out, w)

  return k(idx, deep_t, wide_t)


def _tc_mlp(emb_t, dense_t, wide2, dwt, db, w1e, w1d, b1, w2, b2, w3, b3,
            wwt, wb, bias):
  c0 = (((0,), (0,)), ((), ()))

  def body(emb_ref, dense_ref, ws_ref, dwt_ref, db_ref, w1e_ref, w1d_ref,
           b1_ref, w2_ref, b2_ref, w3_ref, b3_ref, wwt_ref, wb_ref,
           bias_ref, out_ref):
    bf = jnp.bfloat16
    dense_blk = dense_ref[...]
    dd = lax.dot_general(dense_blk, dwt_ref[...], c0,
                         preferred_element_type=jnp.float32) + db_ref[...]
    h1 = lax.dot_general(emb_ref[...].astype(bf), w1e_ref[...].astype(bf),
                         c0, preferred_element_type=jnp.float32)
    h1 = h1 + jnp.dot(dd, w1d_ref[...],
                      preferred_element_type=jnp.float32) + b1_ref[...]
    h1 = jnp.maximum(h1, 0.0)
    h2 = jnp.maximum(
        jnp.dot(h1.astype(bf), w2_ref[...].astype(bf),
                preferred_element_type=jnp.float32)
        + b2_ref[...], 0.0)
    h3 = jnp.maximum(
        jnp.dot(h2.astype(bf), w3_ref[...].astype(bf),
                preferred_element_type=jnp.float32)
        + b3_ref[...], 0.0)
    wd = lax.dot_general(dense_blk, wwt_ref[...], c0,
                         preferred_element_type=jnp.float32) + wb_ref[...]
    ws = lax.dot_general(ws_ref[...], jnp.ones((F, 1), jnp.float32), c0,
                         preferred_element_type=jnp.float32)
    out_ref[...] = bias_ref[...] + ws + wd + h3

  full = lambda a: pl.BlockSpec(a.shape, lambda i: (0,) * a.ndim)
  col_spec = lambda rows: pl.BlockSpec((rows, BT), lambda i: (0, i))
  return pl.pallas_call(
      body,
      grid=(B // BT,),
      in_specs=[
          col_spec(F * D),
          col_spec(ND),
          col_spec(F),
          full(dwt), full(db), full(w1e), full(w1d), full(b1),
          full(w2), full(b2), full(w3), full(b3),
          full(wwt), full(wb), full(bias),
      ],
      out_specs=pl.BlockSpec((BT, 1), lambda i: (i, 0)),
      out_shape=jax.ShapeDtypeStruct((B, 1), jnp.float32),
  )(emb_t, dense_t, wide2, dwt, db, w1e, w1d, b1, w2, b2, w3, b3,
    wwt, wb, bias)


def kernel(sparse_features, dense_features, wide_emb, wide_w, wide_b,
           deep_emb, deep_w, deep_b, W1, b1, W2, b2, W3, b3, bias):
  deep_t = deep_emb.transpose(0, 2, 1).reshape(F * D, V)
  wide_t = wide_emb.reshape(F, V)

  emb_t, wide2 = _sc_gather(sparse_features, deep_t, wide_t)

  return _tc_mlp(
      emb_t,
      dense_features.T,
      wide2,
      deep_w.T,
      deep_b.reshape(1, D),
      W1[:, D:].T,
      W1[:, :D].T,
      b1.reshape(1, 512),
      W2.T,
      b2.reshape(1, 256),
      W3.T,
      b3.reshape(1, 1),
      wide_w.T,
      wide_b.reshape(1, 1),
      bias,
  )
